# baseline (speedup 1.0000x reference)
import functools

import jax
import jax.numpy as jnp
from jax import lax
from jax.experimental import pallas as pl
from jax.experimental.pallas import tpu as pltpu
from jax.experimental.pallas import tpu_sc as plsc

BATCH = 16384
EMB = 32
VOCAB = 1_000_000
NUM_CORES = 2
NUM_SUBCORES = 16
NUM_WORKERS = NUM_CORES * NUM_SUBCORES
RPW = BATCH // NUM_WORKERS

XP_BLOCK = 4096
XP_CHUNK = XP_BLOCK // 4
XP_GRID = -(-VOCAB // XP_BLOCK)
LINES = XP_GRID * XP_CHUNK
PACKED_ROWS = 4 * LINES


def _xpose_body(t_ref, o_ref):
    parts = [
        jnp.transpose(t_ref[:, q * XP_CHUNK:(q + 1) * XP_CHUNK])
        for q in range(4)
    ]
    o_ref[...] = jnp.concatenate(parts, axis=1)


def _xpose_table(tableT):
    return pl.pallas_call(
        _xpose_body,
        grid=(XP_GRID,),
        in_specs=[pl.BlockSpec((EMB, XP_BLOCK), lambda i: (0, i))],
        out_specs=pl.BlockSpec((XP_CHUNK, 4 * EMB), lambda i: (i, 0)),
        out_shape=jax.ShapeDtypeStruct((LINES, 4 * EMB), jnp.float32),
    )(tableT)


_sc_mesh = plsc.VectorSubcoreMesh(core_axis_name="c", subcore_axis_name="s")


@functools.partial(
    pl.kernel,
    mesh=_sc_mesh,
    compiler_params=pltpu.CompilerParams(use_tc_tiling_on_sc=False),
    out_type=jax.ShapeDtypeStruct((BATCH, EMB), jnp.float32),
    scratch_types=[
        pltpu.VMEM((RPW,), jnp.int32),
        pltpu.VMEM((RPW, EMB), jnp.float32),
        pltpu.SemaphoreType.DMA,
    ],
)
def _sc_gather(packed_hbm, idx_hbm, out_hbm, idx_v, rows_v, sem):
    wid = lax.axis_index("s") * NUM_CORES + lax.axis_index("c")
    base = wid * RPW
    pltpu.sync_copy(idx_hbm.at[pl.ds(base, RPW)], idx_v)

    pltpu.async_copy(packed_hbm.at[idx_v], rows_v, sem).wait()
    pltpu.sync_copy(rows_v, out_hbm.at[pl.ds(base, RPW)])


def _remap_body(i_ref, o_ref):
    r = i_ref[...]
    o_ref[...] = ((r >> 12) << 12) + ((r & 1023) << 2) + ((r >> 10) & 3)


def _remap_ids(user_ids):
    return pl.pallas_call(
        _remap_body,
        out_shape=jax.ShapeDtypeStruct((BATCH,), jnp.int32),
    )(user_ids)


def _dot_sigmoid_body(m_ref, u_ref, w_ref, b_ref, o_ref):
    s = jnp.sum(m_ref[...] * u_ref[...], axis=1, keepdims=True)
    o_ref[...] = jax.nn.sigmoid(s * w_ref[0, 0] + b_ref[0])


_TC_BLOCK = 2048


def _tc_dot_sigmoid(movie_emb, uemb, W, b):
    grid = BATCH // _TC_BLOCK
    return pl.pallas_call(
        _dot_sigmoid_body,
        grid=(grid,),
        in_specs=[
            pl.BlockSpec((_TC_BLOCK, EMB), lambda i: (i, 0)),
            pl.BlockSpec((_TC_BLOCK, EMB), lambda i: (i, 0)),
            pl.BlockSpec(memory_space=pltpu.SMEM),
            pl.BlockSpec(memory_space=pltpu.SMEM),
        ],
        out_specs=pl.BlockSpec((_TC_BLOCK, 1), lambda i: (i, 0)),
        out_shape=jax.ShapeDtypeStruct((BATCH, 1), jnp.float32),
    )(movie_emb, uemb, W, b)


@jax.jit
def kernel(user_ids, movie_emb, table, W, b):
    packed = _xpose_table(table.T)
    uemb = _sc_gather(jnp.reshape(packed, (PACKED_ROWS, EMB)),
                      _remap_ids(user_ids.astype(jnp.int32)))
    return _tc_dot_sigmoid(movie_emb, uemb, W, b)


# speedup vs baseline: 1.1603x; 1.1603x over previous
import functools

import jax
import jax.numpy as jnp
from jax import lax
from jax.experimental import pallas as pl
from jax.experimental.pallas import tpu as pltpu
from jax.experimental.pallas import tpu_sc as plsc

BATCH = 16384
EMB = 32
VOCAB = 1_000_000
NUM_CORES = 2
NUM_SUBCORES = 16
NUM_WORKERS = NUM_CORES * NUM_SUBCORES
RPW = BATCH // NUM_WORKERS

XP_BLOCK = 16384
XP_CHUNK = XP_BLOCK // 4
XP_GRID = -(-VOCAB // XP_BLOCK)
LINES = XP_GRID * XP_CHUNK
PACKED_ROWS = 4 * LINES


def _xpose_body(t_ref, o_ref):
    parts = [
        jnp.transpose(t_ref[:, q * XP_CHUNK:(q + 1) * XP_CHUNK])
        for q in range(4)
    ]
    o_ref[...] = jnp.concatenate(parts, axis=1)


def _xpose_table(tableT):
    return pl.pallas_call(
        _xpose_body,
        grid=(XP_GRID,),
        in_specs=[pl.BlockSpec((EMB, XP_BLOCK), lambda i: (0, i))],
        out_specs=pl.BlockSpec((XP_CHUNK, 4 * EMB), lambda i: (i, 0)),
        out_shape=jax.ShapeDtypeStruct((LINES, 4 * EMB), jnp.float32),
        compiler_params=pltpu.CompilerParams(
            dimension_semantics=("parallel",)),
    )(tableT)


_sc_mesh = plsc.VectorSubcoreMesh(core_axis_name="c", subcore_axis_name="s")


@functools.partial(
    pl.kernel,
    mesh=_sc_mesh,
    compiler_params=pltpu.CompilerParams(use_tc_tiling_on_sc=False),
    out_type=jax.ShapeDtypeStruct((BATCH, EMB), jnp.float32),
    scratch_types=[
        pltpu.VMEM((RPW,), jnp.int32),
        pltpu.VMEM((RPW, EMB), jnp.float32),
        pltpu.SemaphoreType.DMA,
    ],
)
def _sc_gather(packed_hbm, idx_hbm, out_hbm, idx_v, rows_v, sem):
    wid = lax.axis_index("s") * NUM_CORES + lax.axis_index("c")
    base = wid * RPW
    pltpu.sync_copy(idx_hbm.at[pl.ds(base, RPW)], idx_v)

    pltpu.async_copy(packed_hbm.at[idx_v], rows_v, sem).wait()
    pltpu.sync_copy(rows_v, out_hbm.at[pl.ds(base, RPW)])


_XPB_LOG2 = XP_BLOCK.bit_length() - 1
_XPC_LOG2 = XP_CHUNK.bit_length() - 1


def _remap_body(i_ref, o_ref):
    r = i_ref[...]
    o_ref[...] = (((r >> _XPB_LOG2) << _XPB_LOG2)
                  + ((r & (XP_CHUNK - 1)) << 2)
                  + ((r >> _XPC_LOG2) & 3))


def _remap_ids(user_ids):
    return pl.pallas_call(
        _remap_body,
        out_shape=jax.ShapeDtypeStruct((BATCH,), jnp.int32),
    )(user_ids)


def _dot_sigmoid_body(m_ref, u_ref, w_ref, b_ref, o_ref):
    s = jnp.sum(m_ref[...] * u_ref[...], axis=1, keepdims=True)
    o_ref[...] = jax.nn.sigmoid(s * w_ref[0, 0] + b_ref[0])


_TC_BLOCK = 2048


def _tc_dot_sigmoid(movie_emb, uemb, W, b):
    grid = BATCH // _TC_BLOCK
    return pl.pallas_call(
        _dot_sigmoid_body,
        grid=(grid,),
        in_specs=[
            pl.BlockSpec((_TC_BLOCK, EMB), lambda i: (i, 0)),
            pl.BlockSpec((_TC_BLOCK, EMB), lambda i: (i, 0)),
            pl.BlockSpec(memory_space=pltpu.SMEM),
            pl.BlockSpec(memory_space=pltpu.SMEM),
        ],
        out_specs=pl.BlockSpec((_TC_BLOCK, 1), lambda i: (i, 0)),
        out_shape=jax.ShapeDtypeStruct((BATCH, 1), jnp.float32),
    )(movie_emb, uemb, W, b)


@jax.jit
def kernel(user_ids, movie_emb, table, W, b):
    packed = _xpose_table(table.T)
    uemb = _sc_gather(jnp.reshape(packed, (PACKED_ROWS, EMB)),
                      _remap_ids(user_ids.astype(jnp.int32)))
    return _tc_dot_sigmoid(movie_emb, uemb, W, b)


# speedup vs baseline: 2.3114x; 1.9920x over previous
import functools

import jax
import jax.numpy as jnp
from jax import lax
from jax.experimental import pallas as pl
from jax.experimental.pallas import tpu as pltpu
from jax.experimental.pallas import tpu_sc as plsc

BATCH = 16384
EMB = 32
VOCAB = 1_000_000
NUM_CORES = 2
NUM_SUBCORES = 16
NUM_WORKERS = NUM_CORES * NUM_SUBCORES
RPW = BATCH // NUM_WORKERS

XP_BLOCK = 16384
XP_CHUNK = XP_BLOCK // 4
XP_GRID = -(-VOCAB // XP_BLOCK)
LINES = XP_GRID * XP_CHUNK
PACKED_ROWS = 4 * LINES


def _xpose_body(t_ref, o_ref):
    stacked = jnp.concatenate(
        [t_ref[:, q * XP_CHUNK:(q + 1) * XP_CHUNK] for q in range(4)], axis=0)
    o_ref[...] = jnp.transpose(stacked)


def _xpose_table(tableT):
    return pl.pallas_call(
        _xpose_body,
        grid=(XP_GRID,),
        in_specs=[pl.BlockSpec((EMB, XP_BLOCK), lambda i: (0, i))],
        out_specs=pl.BlockSpec((XP_CHUNK, 4 * EMB), lambda i: (i, 0)),
        out_shape=jax.ShapeDtypeStruct((LINES, 4 * EMB), jnp.float32),
        compiler_params=pltpu.CompilerParams(
            dimension_semantics=("parallel",)),
    )(tableT)


_sc_mesh = plsc.VectorSubcoreMesh(core_axis_name="c", subcore_axis_name="s")


@functools.partial(
    pl.kernel,
    mesh=_sc_mesh,
    compiler_params=pltpu.CompilerParams(use_tc_tiling_on_sc=False),
    out_type=jax.ShapeDtypeStruct((BATCH, EMB), jnp.float32),
    scratch_types=[
        pltpu.VMEM((RPW,), jnp.int32),
        pltpu.VMEM((RPW, EMB), jnp.float32),
        pltpu.SemaphoreType.DMA,
    ],
)
def _sc_gather(packed_hbm, idx_hbm, out_hbm, idx_v, rows_v, sem):
    wid = lax.axis_index("s") * NUM_CORES + lax.axis_index("c")
    base = wid * RPW
    pltpu.sync_copy(idx_hbm.at[pl.ds(base, RPW)], idx_v)

    pltpu.async_copy(packed_hbm.at[idx_v], rows_v, sem).wait()
    pltpu.sync_copy(rows_v, out_hbm.at[pl.ds(base, RPW)])


_XPB_LOG2 = XP_BLOCK.bit_length() - 1
_XPC_LOG2 = XP_CHUNK.bit_length() - 1


def _remap_body(i_ref, o_ref):
    r = i_ref[...]
    o_ref[...] = (((r >> _XPB_LOG2) << _XPB_LOG2)
                  + ((r & (XP_CHUNK - 1)) << 2)
                  + ((r >> _XPC_LOG2) & 3))


def _remap_ids(user_ids):
    return pl.pallas_call(
        _remap_body,
        out_shape=jax.ShapeDtypeStruct((BATCH,), jnp.int32),
    )(user_ids)


def _dot_sigmoid_body(m_ref, u_ref, w_ref, b_ref, o_ref):
    s = jnp.sum(m_ref[...] * u_ref[...], axis=1, keepdims=True)
    o_ref[...] = jax.nn.sigmoid(s * w_ref[0, 0] + b_ref[0])


_TC_BLOCK = 2048


def _tc_dot_sigmoid(movie_emb, uemb, W, b):
    grid = BATCH // _TC_BLOCK
    return pl.pallas_call(
        _dot_sigmoid_body,
        grid=(grid,),
        in_specs=[
            pl.BlockSpec((_TC_BLOCK, EMB), lambda i: (i, 0)),
            pl.BlockSpec((_TC_BLOCK, EMB), lambda i: (i, 0)),
            pl.BlockSpec(memory_space=pltpu.SMEM),
            pl.BlockSpec(memory_space=pltpu.SMEM),
        ],
        out_specs=pl.BlockSpec((_TC_BLOCK, 1), lambda i: (i, 0)),
        out_shape=jax.ShapeDtypeStruct((BATCH, 1), jnp.float32),
    )(movie_emb, uemb, W, b)


@jax.jit
def kernel(user_ids, movie_emb, table, W, b):
    packed = _xpose_table(table.T)
    uemb = _sc_gather(jnp.reshape(packed, (PACKED_ROWS, EMB)),
                      _remap_ids(user_ids.astype(jnp.int32)))
    return _tc_dot_sigmoid(movie_emb, uemb, W, b)


# speedup vs baseline: 2.6100x; 1.1292x over previous
import functools

import jax
import jax.numpy as jnp
from jax import lax
from jax.experimental import pallas as pl
from jax.experimental.pallas import tpu as pltpu
from jax.experimental.pallas import tpu_sc as plsc

BATCH = 16384
EMB = 32
VOCAB = 1_000_000
NUM_CORES = 2
NUM_SUBCORES = 16
NUM_WORKERS = NUM_CORES * NUM_SUBCORES
RPW = BATCH // NUM_WORKERS

XP_BLOCK = 32768
XP_CHUNK = XP_BLOCK // 4
XP_GRID = -(-VOCAB // XP_BLOCK)
LINES = XP_GRID * XP_CHUNK
PACKED_ROWS = 4 * LINES


def _xpose_body(t_ref, o_ref):
    stacked = jnp.concatenate(
        [t_ref[:, q * XP_CHUNK:(q + 1) * XP_CHUNK] for q in range(4)], axis=0)
    o_ref[...] = jnp.transpose(stacked)


def _xpose_table(tableT):
    return pl.pallas_call(
        _xpose_body,
        grid=(XP_GRID,),
        in_specs=[pl.BlockSpec((EMB, XP_BLOCK), lambda i: (0, i))],
        out_specs=pl.BlockSpec((XP_CHUNK, 4 * EMB), lambda i: (i, 0)),
        out_shape=jax.ShapeDtypeStruct((LINES, 4 * EMB), jnp.float32),
        compiler_params=pltpu.CompilerParams(
            dimension_semantics=("arbitrary",)),
    )(tableT)


_sc_mesh = plsc.VectorSubcoreMesh(core_axis_name="c", subcore_axis_name="s")


@functools.partial(
    pl.kernel,
    mesh=_sc_mesh,
    compiler_params=pltpu.CompilerParams(use_tc_tiling_on_sc=False),
    out_type=jax.ShapeDtypeStruct((BATCH, EMB), jnp.float32),
    scratch_types=[
        pltpu.VMEM((RPW,), jnp.int32),
        pltpu.VMEM((RPW, EMB), jnp.float32),
        pltpu.SemaphoreType.DMA,
    ],
)
def _sc_gather(packed_hbm, idx_hbm, out_hbm, idx_v, rows_v, sem):
    wid = lax.axis_index("s") * NUM_CORES + lax.axis_index("c")
    base = wid * RPW
    pltpu.sync_copy(idx_hbm.at[pl.ds(base, RPW)], idx_v)

    pltpu.async_copy(packed_hbm.at[idx_v], rows_v, sem).wait()
    pltpu.sync_copy(rows_v, out_hbm.at[pl.ds(base, RPW)])


_XPB_LOG2 = XP_BLOCK.bit_length() - 1
_XPC_LOG2 = XP_CHUNK.bit_length() - 1


def _remap_body(i_ref, o_ref):
    r = i_ref[...]
    o_ref[...] = (((r >> _XPB_LOG2) << _XPB_LOG2)
                  + ((r & (XP_CHUNK - 1)) << 2)
                  + ((r >> _XPC_LOG2) & 3))


def _remap_ids(user_ids):
    return pl.pallas_call(
        _remap_body,
        out_shape=jax.ShapeDtypeStruct((BATCH,), jnp.int32),
    )(user_ids)


def _dot_sigmoid_body(m_ref, u_ref, w_ref, b_ref, o_ref):
    s = jnp.sum(m_ref[...] * u_ref[...], axis=1, keepdims=True)
    o_ref[...] = jax.nn.sigmoid(s * w_ref[0, 0] + b_ref[0])


_TC_BLOCK = 8192


def _tc_dot_sigmoid(movie_emb, uemb, W, b):
    grid = BATCH // _TC_BLOCK
    return pl.pallas_call(
        _dot_sigmoid_body,
        grid=(grid,),
        in_specs=[
            pl.BlockSpec((_TC_BLOCK, EMB), lambda i: (i, 0)),
            pl.BlockSpec((_TC_BLOCK, EMB), lambda i: (i, 0)),
            pl.BlockSpec(memory_space=pltpu.SMEM),
            pl.BlockSpec(memory_space=pltpu.SMEM),
        ],
        out_specs=pl.BlockSpec((_TC_BLOCK, 1), lambda i: (i, 0)),
        out_shape=jax.ShapeDtypeStruct((BATCH, 1), jnp.float32),
    )(movie_emb, uemb, W, b)


@jax.jit
def kernel(user_ids, movie_emb, table, W, b):
    packed = _xpose_table(table.T)
    uemb = _sc_gather(jnp.reshape(packed, (PACKED_ROWS, EMB)),
                      _remap_ids(user_ids.astype(jnp.int32)))
    return _tc_dot_sigmoid(movie_emb, uemb, W, b)


# speedup vs baseline: 2.6242x; 1.0054x over previous
import functools

import jax
import jax.numpy as jnp
from jax import lax
from jax.experimental import pallas as pl
from jax.experimental.pallas import tpu as pltpu
from jax.experimental.pallas import tpu_sc as plsc

BATCH = 16384
EMB = 32
VOCAB = 1_000_000
NUM_CORES = 2
NUM_SUBCORES = 16
NUM_WORKERS = NUM_CORES * NUM_SUBCORES
RPW = BATCH // NUM_WORKERS

XP_BLOCK = 65536
XP_CHUNK = XP_BLOCK // 4
XP_GRID = -(-VOCAB // XP_BLOCK)
LINES = XP_GRID * XP_CHUNK
PACKED_ROWS = 4 * LINES


def _xpose_body(t_ref, o_ref):
    stacked = jnp.concatenate(
        [t_ref[:, q * XP_CHUNK:(q + 1) * XP_CHUNK] for q in range(4)], axis=0)
    o_ref[...] = jnp.transpose(stacked)


def _xpose_table(tableT):
    return pl.pallas_call(
        _xpose_body,
        grid=(XP_GRID,),
        in_specs=[pl.BlockSpec((EMB, XP_BLOCK), lambda i: (0, i))],
        out_specs=pl.BlockSpec((XP_CHUNK, 4 * EMB), lambda i: (i, 0)),
        out_shape=jax.ShapeDtypeStruct((LINES, 4 * EMB), jnp.float32),
        compiler_params=pltpu.CompilerParams(
            dimension_semantics=("arbitrary",)),
    )(tableT)


_sc_mesh = plsc.VectorSubcoreMesh(core_axis_name="c", subcore_axis_name="s")


@functools.partial(
    pl.kernel,
    mesh=_sc_mesh,
    compiler_params=pltpu.CompilerParams(use_tc_tiling_on_sc=False),
    out_type=jax.ShapeDtypeStruct((BATCH, EMB), jnp.float32),
    scratch_types=[
        pltpu.VMEM((RPW,), jnp.int32),
        pltpu.VMEM((RPW, EMB), jnp.float32),
        pltpu.SemaphoreType.DMA,
    ],
)
def _sc_gather(packed_hbm, idx_hbm, out_hbm, idx_v, rows_v, sem):
    wid = lax.axis_index("s") * NUM_CORES + lax.axis_index("c")
    base = wid * RPW
    pltpu.sync_copy(idx_hbm.at[pl.ds(base, RPW)], idx_v)

    pltpu.async_copy(packed_hbm.at[idx_v], rows_v, sem).wait()
    pltpu.sync_copy(rows_v, out_hbm.at[pl.ds(base, RPW)])


_XPB_LOG2 = XP_BLOCK.bit_length() - 1
_XPC_LOG2 = XP_CHUNK.bit_length() - 1


def _remap_body(i_ref, o_ref):
    r = i_ref[...]
    o_ref[...] = (((r >> _XPB_LOG2) << _XPB_LOG2)
                  + ((r & (XP_CHUNK - 1)) << 2)
                  + ((r >> _XPC_LOG2) & 3))


def _remap_ids(user_ids):
    return pl.pallas_call(
        _remap_body,
        out_shape=jax.ShapeDtypeStruct((BATCH,), jnp.int32),
    )(user_ids)


def _dot_sigmoid_body(m_ref, u_ref, w_ref, b_ref, o_ref):
    s = jnp.sum(m_ref[...] * u_ref[...], axis=1, keepdims=True)
    o_ref[...] = jax.nn.sigmoid(s * w_ref[0, 0] + b_ref[0])


_TC_BLOCK = 8192


def _tc_dot_sigmoid(movie_emb, uemb, W, b):
    grid = BATCH // _TC_BLOCK
    return pl.pallas_call(
        _dot_sigmoid_body,
        grid=(grid,),
        in_specs=[
            pl.BlockSpec((_TC_BLOCK, EMB), lambda i: (i, 0)),
            pl.BlockSpec((_TC_BLOCK, EMB), lambda i: (i, 0)),
            pl.BlockSpec(memory_space=pltpu.SMEM),
            pl.BlockSpec(memory_space=pltpu.SMEM),
        ],
        out_specs=pl.BlockSpec((_TC_BLOCK, 1), lambda i: (i, 0)),
        out_shape=jax.ShapeDtypeStruct((BATCH, 1), jnp.float32),
    )(movie_emb, uemb, W, b)


@jax.jit
def kernel(user_ids, movie_emb, table, W, b):
    packed = _xpose_table(table.T)
    uemb = _sc_gather(jnp.reshape(packed, (PACKED_ROWS, EMB)),
                      _remap_ids(user_ids.astype(jnp.int32)))
    return _tc_dot_sigmoid(movie_emb, uemb, W, b)
